# baseline (speedup 1.0000x reference)
import jax
import jax.numpy as jnp
from jax.experimental import pallas as pl
from jax.experimental.pallas import tpu as pltpu


_SBLK = 1024


def _add_pe_kernel(x1_ref, x2_ref, pe1_ref, pe2_ref, o_ref):
    o_ref[:, :, : o_ref.shape[2] // 2] = x1_ref[...] + pe1_ref[...][None, :, :]
    o_ref[:, :, o_ref.shape[2] // 2 :] = x2_ref[...] + pe2_ref[...][None, :, :]


def kernel(x, pe_weight):
    B, S, D = x.shape
    H = D // 2
    grid = (S // _SBLK,)
    return pl.pallas_call(
        _add_pe_kernel,
        grid=grid,
        in_specs=[
            pl.BlockSpec((B, _SBLK, H), lambda i: (0, i, 0)),
            pl.BlockSpec((B, _SBLK, H), lambda i: (0, i, 1)),
            pl.BlockSpec((_SBLK, H), lambda i: (i, 0)),
            pl.BlockSpec((_SBLK, H), lambda i: (i, 1)),
        ],
        out_specs=pl.BlockSpec((B, _SBLK, D), lambda i: (0, i, 0)),
        out_shape=jax.ShapeDtypeStruct((B, S, D), x.dtype),
        compiler_params=pltpu.CompilerParams(
            dimension_semantics=("parallel",),
        ),
    )(x, x, pe_weight, pe_weight)
